# Initial kernel scaffold; baseline (speedup 1.0000x reference)
import jax
import jax.numpy as jnp
from jax.experimental import pallas as pl


def kernel(x, edge_index, edge_weight, W, b):
    raise NotImplementedError("write your pallas kernel here")


# speedup vs baseline: 4.5009x; 4.5009x over previous
import functools

import jax
import jax.numpy as jnp
from jax import lax
from jax.experimental import pallas as pl
from jax.experimental.pallas import tpu as pltpu
from jax.experimental.pallas import tpu_sc as plsc

N_NODES = 10000
N_PAD = 10240
N_EDGES = 320000
D = 128

NC, NS, L = 2, 16, 16
NW = NC * NS
EPW = N_EDGES // NW
C = 80
NCHUNK = EPW // C
ROWS_PT = N_PAD // NS
ZR = 128


def _sc_aggregate(x, src, dst, ew):
    mesh = plsc.VectorSubcoreMesh(core_axis_name="c", subcore_axis_name="s")

    @functools.partial(
        pl.kernel,
        out_type=jax.ShapeDtypeStruct((NC, N_PAD, D), jnp.float32),
        mesh=mesh,
        scratch_types=[
            pltpu.VMEM((C,), jnp.int32),
            pltpu.VMEM((C,), jnp.int32),
            pltpu.VMEM((C,), jnp.float32),
            pltpu.VMEM((C, D), jnp.float32),
            pltpu.VMEM((ZR, D), jnp.float32),
            pltpu.VMEM_SHARED((N_PAD, D), jnp.float32),
            pltpu.SemaphoreType.DMA,
        ],
    )
    def agg(x_hbm, src_hbm, dst_hbm, ew_hbm, out_hbm,
            idx_v, dst_v, ew_v, rows_v, zrows_v, acc, sem):
        cid = lax.axis_index("c")
        sid = lax.axis_index("s")
        wid = cid * NS + sid

        zero16 = jnp.zeros((L,), jnp.float32)

        def zstore(i, _):
            for k in range(D // L):
                zrows_v[i, pl.ds(k * L, L)] = zero16
            return 0

        lax.fori_loop(0, ZR, zstore, 0)
        rbase = sid * ROWS_PT
        for j in range(ROWS_PT // ZR):
            pltpu.sync_copy(zrows_v, acc.at[pl.ds(rbase + j * ZR, ZR)])
        plsc.subcore_barrier()

        ebase = wid * EPW

        def echunk(g, _):
            b = pl.multiple_of(ebase + g * C, 8)
            pltpu.sync_copy(src_hbm.at[pl.ds(b, C)], idx_v)
            pltpu.sync_copy(dst_hbm.at[pl.ds(b, C)], dst_v)
            pltpu.sync_copy(ew_hbm.at[pl.ds(b, C)], ew_v)
            pltpu.async_copy(x_hbm.at[idx_v], rows_v, sem).wait()

            def scale(gg, _):
                wvec = ew_v[pl.ds(gg * L, L)]
                for j in range(L):
                    w = wvec[j]
                    e = gg * L + j
                    for k in range(D // L):
                        sl = pl.ds(k * L, L)
                        rows_v[e, sl] = rows_v[e, sl] * w
                return 0

            lax.fori_loop(0, C // L, scale, 0)
            pltpu.sync_copy(rows_v, acc.at[dst_v], add=True)
            return 0

        lax.fori_loop(0, NCHUNK, echunk, 0)
        plsc.subcore_barrier()

        pltpu.sync_copy(acc.at[pl.ds(rbase, ROWS_PT)],
                        out_hbm.at[cid, pl.ds(rbase, ROWS_PT)])

    return agg(x, src, dst, ew)


def _tc_combine(parts, W, b):
    R = 1024

    def body(p_ref, w_ref, b_ref, o_ref):
        s = p_ref[0] + p_ref[1]
        o_ref[...] = (
            jnp.dot(s, w_ref[...], preferred_element_type=jnp.float32)
            + b_ref[...]
        )

    return pl.pallas_call(
        body,
        grid=(N_PAD // R,),
        in_specs=[
            pl.BlockSpec((NC, R, D), lambda i: (0, i, 0)),
            pl.BlockSpec((D, D), lambda i: (0, 0)),
            pl.BlockSpec((1, D), lambda i: (0, 0)),
        ],
        out_specs=pl.BlockSpec((R, D), lambda i: (i, 0)),
        out_shape=jax.ShapeDtypeStruct((N_PAD, D), jnp.float32),
    )(parts, W, b.reshape(1, D))


def kernel(x, edge_index, edge_weight, W, b):
    src = edge_index[0].astype(jnp.int32)
    dst = edge_index[1].astype(jnp.int32)
    parts = _sc_aggregate(x, src, dst, edge_weight)
    return _tc_combine(parts, W, b)[:N_NODES]


# speedup vs baseline: 6.2954x; 1.3987x over previous
import functools

import jax
import jax.numpy as jnp
from jax import lax
from jax.experimental import pallas as pl
from jax.experimental.pallas import tpu as pltpu
from jax.experimental.pallas import tpu_sc as plsc

N_NODES = 10000
N_PAD = 10240
N_EDGES = 320000
D = 128

NC, NS, L = 2, 16, 16
NW = NC * NS
EPW = N_EDGES // NW
C = 80
NCHUNK = EPW // C
ROWS_PT = N_PAD // NS
ZR = 128


def _sc_aggregate(x, src, dst, ew):
    mesh = plsc.VectorSubcoreMesh(core_axis_name="c", subcore_axis_name="s")

    @functools.partial(
        pl.kernel,
        out_type=jax.ShapeDtypeStruct((NC, N_PAD, D), jnp.float32),
        mesh=mesh,
        scratch_types=[
            pltpu.VMEM((2, C), jnp.int32),
            pltpu.VMEM((2, C), jnp.int32),
            pltpu.VMEM((2, C), jnp.float32),
            pltpu.VMEM((2, C, D), jnp.float32),
            pltpu.VMEM((ZR, D), jnp.float32),
            pltpu.VMEM_SHARED((N_PAD, D), jnp.float32),
            pltpu.SemaphoreType.DMA,
            pltpu.SemaphoreType.DMA,
            pltpu.SemaphoreType.DMA,
            pltpu.SemaphoreType.DMA,
        ],
    )
    def agg(x_hbm, src_hbm, dst_hbm, ew_hbm, out_hbm,
            idx_v, dst_v, ew_v, rows_v, zrows_v, acc,
            sg0, sg1, ss0, ss1):
        semG = (sg0, sg1)
        semS = (ss0, ss1)
        cid = lax.axis_index("c")
        sid = lax.axis_index("s")
        wid = cid * NS + sid

        zero16 = jnp.zeros((L,), jnp.float32)

        def zstore(i, _):
            for k in range(D // L):
                zrows_v[i, pl.ds(k * L, L)] = zero16
            return 0

        lax.fori_loop(0, ZR, zstore, 0)
        rbase = sid * ROWS_PT
        for j in range(ROWS_PT // ZR):
            pltpu.sync_copy(zrows_v, acc.at[pl.ds(rbase + j * ZR, ZR)])
        plsc.subcore_barrier()

        ebase = wid * EPW

        def load_indices(g, buf):
            b = pl.multiple_of(ebase + g * C, 8)
            pltpu.sync_copy(src_hbm.at[pl.ds(b, C)], idx_v.at[buf])
            pltpu.sync_copy(dst_hbm.at[pl.ds(b, C)], dst_v.at[buf])
            pltpu.sync_copy(ew_hbm.at[pl.ds(b, C)], ew_v.at[buf])

        def start_gather(buf):
            pltpu.async_copy(x_hbm.at[idx_v.at[buf]], rows_v.at[buf], semG[buf])

        def wait_gather(buf):
            pltpu.make_async_copy(
                x_hbm.at[idx_v.at[buf]], rows_v.at[buf], semG[buf]).wait()

        def start_scatter(buf):
            pltpu.async_copy(rows_v.at[buf], acc.at[dst_v.at[buf]], semS[buf],
                             add=True)

        def wait_scatter(buf):
            pltpu.make_async_copy(
                rows_v.at[buf], acc.at[dst_v.at[buf]], semS[buf]).wait()

        def scale(buf):
            def sbody(gg, _):
                wvec = ew_v[buf, pl.ds(gg * L, L)]
                for j in range(L):
                    w = wvec[j]
                    e = gg * L + j
                    for k in range(D // L):
                        sl = pl.ds(k * L, L)
                        rows_v[buf, e, sl] = rows_v[buf, e, sl] * w
                return 0

            lax.fori_loop(0, C // L, sbody, 0)

        load_indices(0, 0)
        start_gather(0)

        @pl.loop(0, NCHUNK + 1, step=2)
        def _(g0):
            for bb in range(2):
                g = g0 + bb
                nb = 1 - bb

                @pl.when(g < NCHUNK)
                def _():
                    @pl.when(jnp.logical_and(g + 1 < NCHUNK, g >= 1))
                    def _():
                        wait_scatter(nb)

                    @pl.when(g + 1 < NCHUNK)
                    def _():
                        load_indices(g + 1, nb)
                        start_gather(nb)

                    wait_gather(bb)
                    scale(bb)
                    start_scatter(bb)

        wait_scatter(0)
        wait_scatter(1)
        plsc.subcore_barrier()

        pltpu.sync_copy(acc.at[pl.ds(rbase, ROWS_PT)],
                        out_hbm.at[cid, pl.ds(rbase, ROWS_PT)])

    return agg(x, src, dst, ew)


def _tc_combine(parts, W, b):
    R = 1024

    def body(p_ref, w_ref, b_ref, o_ref):
        s = p_ref[0] + p_ref[1]
        o_ref[...] = (
            jnp.dot(s, w_ref[...], preferred_element_type=jnp.float32)
            + b_ref[...]
        )

    return pl.pallas_call(
        body,
        grid=(N_PAD // R,),
        in_specs=[
            pl.BlockSpec((NC, R, D), lambda i: (0, i, 0)),
            pl.BlockSpec((D, D), lambda i: (0, 0)),
            pl.BlockSpec((1, D), lambda i: (0, 0)),
        ],
        out_specs=pl.BlockSpec((R, D), lambda i: (i, 0)),
        out_shape=jax.ShapeDtypeStruct((N_PAD, D), jnp.float32),
    )(parts, W, b.reshape(1, D))


def kernel(x, edge_index, edge_weight, W, b):
    src = edge_index[0].astype(jnp.int32)
    dst = edge_index[1].astype(jnp.int32)
    parts = _sc_aggregate(x, src, dst, edge_weight)
    return _tc_combine(parts, W, b)[:N_NODES]


# speedup vs baseline: 9.2551x; 1.4701x over previous
import functools

import jax
import jax.numpy as jnp
from jax import lax
from jax.experimental import pallas as pl
from jax.experimental.pallas import tpu as pltpu
from jax.experimental.pallas import tpu_sc as plsc

N_NODES = 10000
N_PAD = 10240
N_EDGES = 320000
D = 128

NC, NS, L = 2, 16, 16
NW = NC * NS
EPW = N_EDGES // NW
C = 80
NCHUNK = EPW // C
NBUF = 2
ROWS_PT = N_PAD // NS


def _sc_aggregate(x, src, dst, ew):
    mesh = plsc.VectorSubcoreMesh(core_axis_name="c", subcore_axis_name="s")

    @functools.partial(
        pl.kernel,
        out_type=jax.ShapeDtypeStruct((NC, N_PAD, D), jnp.float32),
        mesh=mesh,
        scratch_types=[
            pltpu.VMEM((EPW,), jnp.int32),
            pltpu.VMEM((EPW,), jnp.float32),
            pltpu.VMEM((NBUF, C), jnp.int32),
            pltpu.VMEM((NBUF, C, D), jnp.float32),
            pltpu.VMEM_SHARED((N_PAD, D), jnp.float32),
            pltpu.SemaphoreType.DMA,
            pltpu.SemaphoreType.DMA,
            pltpu.SemaphoreType.DMA,
            pltpu.SemaphoreType.DMA,
        ],
    )
    def agg(x_hbm, src_hbm, dst_hbm, ew_hbm, out_hbm,
            idx_v, ew_v, dst_v, rows_v, acc,
            sg0, sg1, ss0, ss1):
        semG = (sg0, sg1)
        semS = (ss0, ss1)
        cid = lax.axis_index("c")
        sid = lax.axis_index("s")
        wid = cid * NS + sid

        zero16 = jnp.zeros((L,), jnp.float32)

        def zstore(i, _):
            for k in range(D // L):
                rows_v[0, i, pl.ds(k * L, L)] = zero16
            return 0

        lax.fori_loop(0, C, zstore, 0)
        rbase = sid * ROWS_PT
        for j in range(ROWS_PT // C):
            pltpu.sync_copy(rows_v.at[0], acc.at[pl.ds(rbase + j * C, C)])
        plsc.subcore_barrier()

        ebase = pl.multiple_of(wid * EPW, 8)
        pltpu.sync_copy(src_hbm.at[pl.ds(ebase, EPW)], idx_v)
        pltpu.sync_copy(ew_hbm.at[pl.ds(ebase, EPW)], ew_v)

        def load_dst(g, buf):
            b = pl.multiple_of(wid * EPW + g * C, 8)
            pltpu.sync_copy(dst_hbm.at[pl.ds(b, C)], dst_v.at[buf])

        def start_gather(g, buf):
            gb = pl.multiple_of(g * C, 8)
            pltpu.async_copy(x_hbm.at[idx_v.at[pl.ds(gb, C)]], rows_v.at[buf],
                             semG[buf])

        def wait_gather(buf):
            pltpu.make_async_copy(
                x_hbm.at[idx_v.at[pl.ds(0, C)]], rows_v.at[buf],
                semG[buf]).wait()

        def start_scatter(buf):
            pltpu.async_copy(rows_v.at[buf], acc.at[dst_v.at[buf]], semS[buf],
                             add=True)

        def wait_scatter(buf):
            pltpu.make_async_copy(
                rows_v.at[buf], acc.at[dst_v.at[buf]], semS[buf]).wait()

        def scale(g, buf):
            def sbody(gg, _):
                wvec = ew_v[pl.ds(g * C + gg * L, L)]
                for j in range(L):
                    w = wvec[j]
                    e = gg * L + j
                    for k in range(D // L):
                        sl = pl.ds(k * L, L)
                        rows_v[buf, e, sl] = rows_v[buf, e, sl] * w
                return 0

            lax.fori_loop(0, C // L, sbody, 0)

        load_dst(0, 0)
        start_gather(0, 0)

        @pl.loop(0, NCHUNK + 1, step=NBUF)
        def _(g0):
            for bb in range(NBUF):
                g = g0 + bb
                nb = 1 - bb

                @pl.when(g < NCHUNK)
                def _():
                    @pl.when(g + 1 < NCHUNK)
                    def _():
                        @pl.when(g >= 1)
                        def _():
                            wait_scatter(nb)

                        load_dst(g + 1, nb)
                        start_gather(g + 1, nb)

                    wait_gather(bb)
                    scale(g, bb)
                    start_scatter(bb)

        for buf in range(NBUF):
            wait_scatter(buf)
        plsc.subcore_barrier()

        pltpu.sync_copy(acc.at[pl.ds(rbase, ROWS_PT)],
                        out_hbm.at[cid, pl.ds(rbase, ROWS_PT)])

    return agg(x, src, dst, ew)


def _tc_combine(parts, W, b):
    R = 1024

    def body(p_ref, w_ref, b_ref, o_ref):
        s = p_ref[0] + p_ref[1]
        o_ref[...] = (
            jnp.dot(s, w_ref[...], preferred_element_type=jnp.float32)
            + b_ref[...]
        )

    return pl.pallas_call(
        body,
        grid=(N_PAD // R,),
        in_specs=[
            pl.BlockSpec((NC, R, D), lambda i: (0, i, 0)),
            pl.BlockSpec((D, D), lambda i: (0, 0)),
            pl.BlockSpec((1, D), lambda i: (0, 0)),
        ],
        out_specs=pl.BlockSpec((R, D), lambda i: (i, 0)),
        out_shape=jax.ShapeDtypeStruct((N_PAD, D), jnp.float32),
    )(parts, W, b.reshape(1, D))


def kernel(x, edge_index, edge_weight, W, b):
    src = edge_index[0].astype(jnp.int32)
    dst = edge_index[1].astype(jnp.int32)
    parts = _sc_aggregate(x, src, dst, edge_weight)
    return _tc_combine(parts, W, b)[:N_NODES]
